# baseline (speedup 1.0000x reference)
import jax
import jax.numpy as jnp
from jax import lax
from jax.experimental import pallas as pl
from jax.experimental.pallas import tpu as pltpu
from jax.experimental.pallas import tpu_sc as plsc

N = 10000
D = 128
E = 320000
NC = 2
NS = 16
NPAD = 10240
ROWS_PER_TILE = NPAD // NS
CHUNK = 80
CHUNKS_PER_TILE = E // (NC * NS * CHUNK)
NGROUP = 5
GCHUNKS = CHUNKS_PER_TILE // NGROUP
ZROWS = 8
NBUF = 3


def _sc_agg_body(feat_hbm, src_hbm, dst_hbm, psum_hbm, pcnt_hbm,
                 acc_sh, cnt_sh, sidx, didx, rows0, rows1, rows2,
                 ones_v, zrows, zcnt, semg, sems):
    rows = (rows0, rows1, rows2)
    c = lax.axis_index("core")
    s = lax.axis_index("subcore")
    t = c * NS + s

    @pl.loop(0, CHUNK // 16)
    def _(k):
        ones_v[pl.ds(k * 16, 16)] = jnp.ones((16,), jnp.float32)

    @pl.loop(0, ZROWS)
    def _(r):
        @pl.loop(0, D // 16)
        def _(k):
            zrows[r, pl.ds(k * 16, 16)] = jnp.zeros((16,), jnp.float32)

    @pl.loop(0, ROWS_PER_TILE // 16)
    def _(k):
        zcnt[pl.ds(k * 16, 16)] = jnp.zeros((16,), jnp.float32)

    @pl.loop(0, ROWS_PER_TILE // ZROWS)
    def _(j):
        off = s * ROWS_PER_TILE + j * ZROWS
        pltpu.sync_copy(zrows, acc_sh.at[pl.ds(off, ZROWS)])
    pltpu.sync_copy(zcnt, cnt_sh.at[pl.ds(s * ROWS_PER_TILE, ROWS_PER_TILE)])

    plsc.subcore_barrier()

    @pl.loop(0, NGROUP)
    def _(g):
        plane = t * NGROUP + g
        pltpu.sync_copy(src_hbm.at[plane], sidx)
        pltpu.sync_copy(dst_hbm.at[plane], didx)

        for b in range(NBUF):
            pltpu.async_copy(feat_hbm.at[sidx.at[b]], rows[b], semg.at[b])

        @pl.loop(0, GCHUNKS, step=NBUF)
        def _(i):
            for b in range(NBUF):
                j = i + b

                @pl.when(j < GCHUNKS)
                def _():
                    pltpu.make_async_copy(feat_hbm.at[sidx.at[j]], rows[b],
                                          semg.at[b]).wait()
                    pltpu.async_copy(rows[b], acc_sh.at[didx.at[j]],
                                     sems.at[b], add=True)
                    pltpu.sync_copy(ones_v, cnt_sh.at[didx.at[j]], add=True)

                @pl.when(j + NBUF < GCHUNKS)
                def _():
                    pltpu.make_async_copy(rows[b], acc_sh.at[didx.at[j]],
                                          sems.at[b]).wait()
                    pltpu.async_copy(feat_hbm.at[sidx.at[j + NBUF]], rows[b],
                                     semg.at[b])

        for b in range(NBUF):
            pltpu.make_async_copy(rows[b], acc_sh.at[didx.at[0]],
                                  sems.at[b]).wait()

    plsc.subcore_barrier()

    off = s * ROWS_PER_TILE
    pltpu.sync_copy(acc_sh.at[pl.ds(off, ROWS_PER_TILE)],
                    psum_hbm.at[c, pl.ds(off, ROWS_PER_TILE)])
    pltpu.sync_copy(cnt_sh.at[pl.ds(off, ROWS_PER_TILE)],
                    pcnt_hbm.at[c, pl.ds(off, ROWS_PER_TILE)])


def _sc_agg(feat, src, dst):
    mesh = plsc.VectorSubcoreMesh(core_axis_name="core",
                                  subcore_axis_name="subcore")
    f = pl.kernel(
        _sc_agg_body,
        out_type=[
            jax.ShapeDtypeStruct((NC, NPAD, D), jnp.float32),
            jax.ShapeDtypeStruct((NC, NPAD), jnp.float32),
        ],
        mesh=mesh,
        scratch_types=[
            pltpu.VMEM_SHARED((NPAD, D), jnp.float32),
            pltpu.VMEM_SHARED((NPAD,), jnp.float32),
            pltpu.VMEM((GCHUNKS, CHUNK), jnp.int32),
            pltpu.VMEM((GCHUNKS, CHUNK), jnp.int32),
            pltpu.VMEM((CHUNK, D), jnp.float32),
            pltpu.VMEM((CHUNK, D), jnp.float32),
            pltpu.VMEM((CHUNK, D), jnp.float32),
            pltpu.VMEM((CHUNK,), jnp.float32),
            pltpu.VMEM((ZROWS, D), jnp.float32),
            pltpu.VMEM((ROWS_PER_TILE,), jnp.float32),
            pltpu.SemaphoreType.DMA((NBUF,)),
            pltpu.SemaphoreType.DMA((NBUF,)),
        ],
    )
    return f(feat, src, dst)


def _tc_body(ps_ref, pc_ref, w_ref, o_ref):
    p = ps_ref[...]
    ssum = p[0] + p[1]
    cc = pc_ref[...]
    deg = jnp.maximum(cc[0] + cc[1], 1.0)
    h = ssum / deg
    o_ref[...] = lax.dot_general(h, w_ref[...], (((1,), (1,)), ((), ())),
                                 preferred_element_type=jnp.float32)


def _tc_finish(psum, pcnt3, w):
    blk = 1000
    return pl.pallas_call(
        _tc_body,
        grid=(N // blk,),
        in_specs=[
            pl.BlockSpec((NC, blk, D), lambda i: (0, i, 0)),
            pl.BlockSpec((NC, blk, 1), lambda i: (0, i, 0)),
            pl.BlockSpec((D, D), lambda i: (0, 0)),
        ],
        out_specs=pl.BlockSpec((blk, D), lambda i: (i, 0)),
        out_shape=jax.ShapeDtypeStruct((N, D), jnp.float32),
    )(psum, pcnt3, w)


def kernel(feat, edge_index, W_neigh):
    src = edge_index[0].reshape(NC * NS * NGROUP, GCHUNKS, CHUNK)
    dst = edge_index[1].reshape(NC * NS * NGROUP, GCHUNKS, CHUNK)
    psum, pcnt = _sc_agg(feat, src, dst)
    return _tc_finish(psum, pcnt.reshape(NC, NPAD, 1), W_neigh)


# speedup vs baseline: 1.0177x; 1.0177x over previous
import jax
import jax.numpy as jnp
from jax import lax
from jax.experimental import pallas as pl
from jax.experimental.pallas import tpu as pltpu
from jax.experimental.pallas import tpu_sc as plsc

N = 10000
D = 128
E = 320000
NC = 2
NS = 16
NPAD = 10240
ROWS_PER_TILE = NPAD // NS
CHUNK = 80
CHUNKS_PER_TILE = E // (NC * NS * CHUNK)
NGROUP = 5
GCHUNKS = CHUNKS_PER_TILE // NGROUP
ZROWS = 16
NBUF = 3


def _sc_agg_body(feat_hbm, src_hbm, dst_hbm, psum_hbm, pcnt_hbm,
                 acc_sh, cnt_sh, sidx, didx, rows0, rows1, rows2,
                 ones_v, zrows, zcnt, semg, sems):
    rows = (rows0, rows1, rows2)
    c = lax.axis_index("core")
    s = lax.axis_index("subcore")
    t = c * NS + s

    @pl.loop(0, CHUNK // 16)
    def _(k):
        ones_v[pl.ds(k * 16, 16)] = jnp.ones((16,), jnp.float32)

    @pl.loop(0, ZROWS)
    def _(r):
        @pl.loop(0, D // 16)
        def _(k):
            zrows[r, pl.ds(k * 16, 16)] = jnp.zeros((16,), jnp.float32)

    @pl.loop(0, ROWS_PER_TILE // 16)
    def _(k):
        zcnt[pl.ds(k * 16, 16)] = jnp.zeros((16,), jnp.float32)

    @pl.loop(0, ROWS_PER_TILE // ZROWS)
    def _(j):
        off = s * ROWS_PER_TILE + j * ZROWS
        pltpu.sync_copy(zrows, acc_sh.at[pl.ds(off, ZROWS)])
    pltpu.sync_copy(zcnt, cnt_sh.at[pl.ds(s * ROWS_PER_TILE, ROWS_PER_TILE)])

    plsc.subcore_barrier()

    @pl.loop(0, NGROUP)
    def _(g):
        plane = t * NGROUP + g
        pltpu.sync_copy(src_hbm.at[plane], sidx)
        pltpu.sync_copy(dst_hbm.at[plane], didx)

        for b in range(NBUF):
            pltpu.async_copy(feat_hbm.at[sidx.at[b]], rows[b], semg.at[b])

        @pl.loop(0, GCHUNKS, step=NBUF)
        def _(i):
            for b in range(NBUF):
                j = i + b

                @pl.when(j < GCHUNKS)
                def _():
                    pltpu.make_async_copy(feat_hbm.at[sidx.at[j]], rows[b],
                                          semg.at[b]).wait()
                    pltpu.async_copy(rows[b], acc_sh.at[didx.at[j]],
                                     sems.at[b], add=True)
                    pltpu.sync_copy(ones_v, cnt_sh.at[didx.at[j]], add=True)

                @pl.when(j + NBUF < GCHUNKS)
                def _():
                    pltpu.make_async_copy(rows[b], acc_sh.at[didx.at[j]],
                                          sems.at[b]).wait()
                    pltpu.async_copy(feat_hbm.at[sidx.at[j + NBUF]], rows[b],
                                     semg.at[b])

        for b in range(NBUF):
            pltpu.make_async_copy(rows[b], acc_sh.at[didx.at[0]],
                                  sems.at[b]).wait()

    plsc.subcore_barrier()

    off = s * ROWS_PER_TILE
    pltpu.sync_copy(acc_sh.at[pl.ds(off, ROWS_PER_TILE)],
                    psum_hbm.at[c, pl.ds(off, ROWS_PER_TILE)])
    pltpu.sync_copy(cnt_sh.at[pl.ds(off, ROWS_PER_TILE)],
                    pcnt_hbm.at[c, pl.ds(off, ROWS_PER_TILE)])


def _sc_agg(feat, src, dst):
    mesh = plsc.VectorSubcoreMesh(core_axis_name="core",
                                  subcore_axis_name="subcore")
    f = pl.kernel(
        _sc_agg_body,
        out_type=[
            jax.ShapeDtypeStruct((NC, NPAD, D), jnp.float32),
            jax.ShapeDtypeStruct((NC, NPAD), jnp.float32),
        ],
        mesh=mesh,
        scratch_types=[
            pltpu.VMEM_SHARED((NPAD, D), jnp.float32),
            pltpu.VMEM_SHARED((NPAD,), jnp.float32),
            pltpu.VMEM((GCHUNKS, CHUNK), jnp.int32),
            pltpu.VMEM((GCHUNKS, CHUNK), jnp.int32),
            pltpu.VMEM((CHUNK, D), jnp.float32),
            pltpu.VMEM((CHUNK, D), jnp.float32),
            pltpu.VMEM((CHUNK, D), jnp.float32),
            pltpu.VMEM((CHUNK,), jnp.float32),
            pltpu.VMEM((ZROWS, D), jnp.float32),
            pltpu.VMEM((ROWS_PER_TILE,), jnp.float32),
            pltpu.SemaphoreType.DMA((NBUF,)),
            pltpu.SemaphoreType.DMA((NBUF,)),
        ],
    )
    return f(feat, src, dst)


def _tc_body(ps_ref, pc_ref, w_ref, o_ref):
    p = ps_ref[...]
    ssum = p[0] + p[1]
    cc = pc_ref[...]
    deg = jnp.maximum(cc[0] + cc[1], 1.0)
    h = ssum / deg
    o_ref[...] = lax.dot_general(h, w_ref[...], (((1,), (1,)), ((), ())),
                                 preferred_element_type=jnp.float32)


def _tc_finish(psum, pcnt3, w):
    blk = 1000
    return pl.pallas_call(
        _tc_body,
        grid=(N // blk,),
        in_specs=[
            pl.BlockSpec((NC, blk, D), lambda i: (0, i, 0)),
            pl.BlockSpec((NC, blk, 1), lambda i: (0, i, 0)),
            pl.BlockSpec((D, D), lambda i: (0, 0)),
        ],
        out_specs=pl.BlockSpec((blk, D), lambda i: (i, 0)),
        out_shape=jax.ShapeDtypeStruct((N, D), jnp.float32),
    )(psum, pcnt3, w)


def kernel(feat, edge_index, W_neigh):
    src = edge_index[0].reshape(NC * NS * NGROUP, GCHUNKS, CHUNK)
    dst = edge_index[1].reshape(NC * NS * NGROUP, GCHUNKS, CHUNK)
    psum, pcnt = _sc_agg(feat, src, dst)
    return _tc_finish(psum, pcnt.reshape(NC, NPAD, 1), W_neigh)


# speedup vs baseline: 1.0910x; 1.0721x over previous
import jax
import jax.numpy as jnp
from jax import lax
from jax.experimental import pallas as pl
from jax.experimental.pallas import tpu as pltpu
from jax.experimental.pallas import tpu_sc as plsc

N = 10000
D = 128
E = 320000
NC = 2
NS = 16
NPAD = 10240
ROWS_PER_TILE = NPAD // NS
CHUNK = 80
CHUNKS_PER_TILE = E // (NC * NS * CHUNK)
NGROUP = 5
GCHUNKS = CHUNKS_PER_TILE // NGROUP
ZROWS = 16
NBUF = 3


def _sc_agg_body(feat_hbm, ei_hbm, psum_hbm, pcnt_hbm,
                 acc_sh, cnt_sh, sidx, didx, rows0, rows1, rows2,
                 ones_v, zrows, zcnt, semg, sems):
    rows = (rows0, rows1, rows2)
    c = lax.axis_index("core")
    s = lax.axis_index("subcore")
    t = c * NS + s

    @pl.loop(0, CHUNK // 16)
    def _(k):
        ones_v[pl.ds(k * 16, 16)] = jnp.ones((16,), jnp.float32)

    @pl.loop(0, ZROWS)
    def _(r):
        @pl.loop(0, D // 16)
        def _(k):
            zrows[r, pl.ds(k * 16, 16)] = jnp.zeros((16,), jnp.float32)

    @pl.loop(0, ROWS_PER_TILE // 16)
    def _(k):
        zcnt[pl.ds(k * 16, 16)] = jnp.zeros((16,), jnp.float32)

    @pl.loop(0, ROWS_PER_TILE // ZROWS)
    def _(j):
        off = s * ROWS_PER_TILE + j * ZROWS
        pltpu.sync_copy(zrows, acc_sh.at[pl.ds(off, ZROWS)])
    pltpu.sync_copy(zcnt, cnt_sh.at[pl.ds(s * ROWS_PER_TILE, ROWS_PER_TILE)])

    plsc.subcore_barrier()

    @pl.loop(0, NGROUP)
    def _(g):
        plane = t * NGROUP + g
        pltpu.sync_copy(ei_hbm.at[0, plane], sidx)
        pltpu.sync_copy(ei_hbm.at[1, plane], didx)

        for b in range(NBUF):
            pltpu.async_copy(feat_hbm.at[sidx.at[b]], rows[b], semg.at[b])

        @pl.loop(0, GCHUNKS, step=NBUF)
        def _(i):
            for b in range(NBUF):
                j = i + b

                @pl.when(j < GCHUNKS)
                def _():
                    pltpu.make_async_copy(feat_hbm.at[sidx.at[j]], rows[b],
                                          semg.at[b]).wait()
                    pltpu.async_copy(rows[b], acc_sh.at[didx.at[j]],
                                     sems.at[b], add=True)
                    pltpu.sync_copy(ones_v, cnt_sh.at[didx.at[j]], add=True)

                @pl.when(j + NBUF < GCHUNKS)
                def _():
                    pltpu.make_async_copy(rows[b], acc_sh.at[didx.at[j]],
                                          sems.at[b]).wait()
                    pltpu.async_copy(feat_hbm.at[sidx.at[j + NBUF]], rows[b],
                                     semg.at[b])

        for b in range(NBUF):
            pltpu.make_async_copy(rows[b], acc_sh.at[didx.at[0]],
                                  sems.at[b]).wait()

    plsc.subcore_barrier()

    off = s * ROWS_PER_TILE
    pltpu.sync_copy(acc_sh.at[pl.ds(off, ROWS_PER_TILE)],
                    psum_hbm.at[c, pl.ds(off, ROWS_PER_TILE)])
    pltpu.sync_copy(cnt_sh.at[pl.ds(off, ROWS_PER_TILE)],
                    pcnt_hbm.at[c, pl.ds(off, ROWS_PER_TILE)])


def _sc_agg(feat, ei):
    mesh = plsc.VectorSubcoreMesh(core_axis_name="core",
                                  subcore_axis_name="subcore")
    f = pl.kernel(
        _sc_agg_body,
        out_type=[
            jax.ShapeDtypeStruct((NC, NPAD, D), jnp.float32),
            jax.ShapeDtypeStruct((NC, NPAD), jnp.float32),
        ],
        mesh=mesh,
        scratch_types=[
            pltpu.VMEM_SHARED((NPAD, D), jnp.float32),
            pltpu.VMEM_SHARED((NPAD,), jnp.float32),
            pltpu.VMEM((GCHUNKS, CHUNK), jnp.int32),
            pltpu.VMEM((GCHUNKS, CHUNK), jnp.int32),
            pltpu.VMEM((CHUNK, D), jnp.float32),
            pltpu.VMEM((CHUNK, D), jnp.float32),
            pltpu.VMEM((CHUNK, D), jnp.float32),
            pltpu.VMEM((CHUNK,), jnp.float32),
            pltpu.VMEM((ZROWS, D), jnp.float32),
            pltpu.VMEM((ROWS_PER_TILE,), jnp.float32),
            pltpu.SemaphoreType.DMA((NBUF,)),
            pltpu.SemaphoreType.DMA((NBUF,)),
        ],
    )
    return f(feat, ei)


def _tc_body(ps_ref, pc_ref, w_ref, o_ref):
    p = ps_ref[...]
    ssum = p[0] + p[1]
    cc = pc_ref[...]
    deg = jnp.maximum(cc[0] + cc[1], 1.0)
    h = (ssum / deg).astype(jnp.bfloat16)
    o_ref[...] = lax.dot_general(h, w_ref[...], (((1,), (1,)), ((), ())),
                                 preferred_element_type=jnp.float32)


def _tc_finish(psum, pcnt3, w):
    blk = 1000
    return pl.pallas_call(
        _tc_body,
        grid=(N // blk,),
        in_specs=[
            pl.BlockSpec((NC, blk, D), lambda i: (0, i, 0)),
            pl.BlockSpec((NC, blk, 1), lambda i: (0, i, 0)),
            pl.BlockSpec((D, D), lambda i: (0, 0)),
        ],
        out_specs=pl.BlockSpec((blk, D), lambda i: (i, 0)),
        out_shape=jax.ShapeDtypeStruct((N, D), jnp.float32),
    )(psum, pcnt3, w.astype(jnp.bfloat16))


def kernel(feat, edge_index, W_neigh):
    ei = edge_index.reshape(2, NC * NS * NGROUP, GCHUNKS, CHUNK)
    psum, pcnt = _sc_agg(feat, ei)
    return _tc_finish(psum, pcnt.reshape(NC, NPAD, 1), W_neigh)


# speedup vs baseline: 1.1071x; 1.0147x over previous
import jax
import jax.numpy as jnp
from jax import lax
from jax.experimental import pallas as pl
from jax.experimental.pallas import tpu as pltpu
from jax.experimental.pallas import tpu_sc as plsc

N = 10000
D = 128
E = 320000
NC = 2
NS = 16
NPAD = 10240
ROWS_PER_TILE = NPAD // NS
CHUNK = 80
CHUNKS_PER_TILE = E // (NC * NS * CHUNK)
NGROUP = 5
GCHUNKS = CHUNKS_PER_TILE // NGROUP
ZROWS = 16
NBUF = 3


def _sc_agg_body(feat_hbm, ei_hbm, psum_hbm, pcnt_hbm,
                 acc_sh, cnt_sh, sidx, didx, rows0, rows1, rows2,
                 ones_v, zrows, zcnt, semg, sems):
    rows = (rows0, rows1, rows2)
    c = lax.axis_index("core")
    s = lax.axis_index("subcore")
    t = c * NS + s

    @pl.loop(0, CHUNK // 16)
    def _(k):
        ones_v[pl.ds(k * 16, 16)] = jnp.ones((16,), jnp.float32)

    @pl.loop(0, ZROWS)
    def _(r):
        @pl.loop(0, D // 16)
        def _(k):
            zrows[r, pl.ds(k * 16, 16)] = jnp.zeros((16,), jnp.float32)

    @pl.loop(0, ROWS_PER_TILE // 16)
    def _(k):
        zcnt[pl.ds(k * 16, 16)] = jnp.zeros((16,), jnp.float32)

    @pl.loop(0, ROWS_PER_TILE // ZROWS)
    def _(j):
        off = s * ROWS_PER_TILE + j * ZROWS
        pltpu.async_copy(zrows, acc_sh.at[pl.ds(off, ZROWS)], semg.at[0])
    pltpu.async_copy(zcnt, cnt_sh.at[pl.ds(s * ROWS_PER_TILE, ROWS_PER_TILE)],
                     semg.at[1])

    @pl.loop(0, ROWS_PER_TILE // ZROWS)
    def _(j):
        pltpu.make_async_copy(zrows, acc_sh.at[pl.ds(s * ROWS_PER_TILE,
                                                     ZROWS)],
                              semg.at[0]).wait()
    pltpu.make_async_copy(zcnt, cnt_sh.at[pl.ds(s * ROWS_PER_TILE,
                                                ROWS_PER_TILE)],
                          semg.at[1]).wait()

    plsc.subcore_barrier()

    @pl.loop(0, NGROUP)
    def _(g):
        plane = t * NGROUP + g
        pltpu.sync_copy(ei_hbm.at[0, plane], sidx)
        pltpu.sync_copy(ei_hbm.at[1, plane], didx)

        for b in range(NBUF):
            pltpu.async_copy(feat_hbm.at[sidx.at[b]], rows[b], semg.at[b])

        @pl.loop(0, GCHUNKS, step=NBUF)
        def _(i):
            for b in range(NBUF):
                j = i + b

                @pl.when(j < GCHUNKS)
                def _():
                    pltpu.make_async_copy(feat_hbm.at[sidx.at[j]], rows[b],
                                          semg.at[b]).wait()
                    pltpu.async_copy(rows[b], acc_sh.at[didx.at[j]],
                                     sems.at[b], add=True)
                    pltpu.sync_copy(ones_v, cnt_sh.at[didx.at[j]], add=True)

                @pl.when(j + NBUF < GCHUNKS)
                def _():
                    pltpu.make_async_copy(rows[b], acc_sh.at[didx.at[j]],
                                          sems.at[b]).wait()
                    pltpu.async_copy(feat_hbm.at[sidx.at[j + NBUF]], rows[b],
                                     semg.at[b])

        for b in range(NBUF):
            pltpu.make_async_copy(rows[b], acc_sh.at[didx.at[0]],
                                  sems.at[b]).wait()

    plsc.subcore_barrier()

    off = s * ROWS_PER_TILE
    pltpu.async_copy(acc_sh.at[pl.ds(off, ROWS_PER_TILE)],
                     psum_hbm.at[c, pl.ds(off, ROWS_PER_TILE)], semg.at[0])
    pltpu.async_copy(cnt_sh.at[pl.ds(off, ROWS_PER_TILE)],
                     pcnt_hbm.at[c, pl.ds(off, ROWS_PER_TILE)], semg.at[1])
    pltpu.make_async_copy(acc_sh.at[pl.ds(off, ROWS_PER_TILE)],
                          psum_hbm.at[c, pl.ds(off, ROWS_PER_TILE)],
                          semg.at[0]).wait()
    pltpu.make_async_copy(cnt_sh.at[pl.ds(off, ROWS_PER_TILE)],
                          pcnt_hbm.at[c, pl.ds(off, ROWS_PER_TILE)],
                          semg.at[1]).wait()


def _sc_agg(feat, ei):
    mesh = plsc.VectorSubcoreMesh(core_axis_name="core",
                                  subcore_axis_name="subcore")
    f = pl.kernel(
        _sc_agg_body,
        out_type=[
            jax.ShapeDtypeStruct((NC, NPAD, D), jnp.float32),
            jax.ShapeDtypeStruct((NC, NPAD), jnp.float32),
        ],
        mesh=mesh,
        scratch_types=[
            pltpu.VMEM_SHARED((NPAD, D), jnp.float32),
            pltpu.VMEM_SHARED((NPAD,), jnp.float32),
            pltpu.VMEM((GCHUNKS, CHUNK), jnp.int32),
            pltpu.VMEM((GCHUNKS, CHUNK), jnp.int32),
            pltpu.VMEM((CHUNK, D), jnp.float32),
            pltpu.VMEM((CHUNK, D), jnp.float32),
            pltpu.VMEM((CHUNK, D), jnp.float32),
            pltpu.VMEM((CHUNK,), jnp.float32),
            pltpu.VMEM((ZROWS, D), jnp.float32),
            pltpu.VMEM((ROWS_PER_TILE,), jnp.float32),
            pltpu.SemaphoreType.DMA((NBUF,)),
            pltpu.SemaphoreType.DMA((NBUF,)),
        ],
    )
    return f(feat, ei)


def _tc_body(ps_ref, pc_ref, w_ref, o_ref):
    p = ps_ref[...]
    ssum = p[0] + p[1]
    cc = pc_ref[...]
    deg = jnp.maximum(cc[0] + cc[1], 1.0)
    h = (ssum / deg).astype(jnp.bfloat16)
    o_ref[...] = lax.dot_general(h, w_ref[...], (((1,), (1,)), ((), ())),
                                 preferred_element_type=jnp.float32)


def _tc_finish(psum, pcnt3, w):
    blk = 1000
    return pl.pallas_call(
        _tc_body,
        grid=(N // blk,),
        in_specs=[
            pl.BlockSpec((NC, blk, D), lambda i: (0, i, 0)),
            pl.BlockSpec((NC, blk, 1), lambda i: (0, i, 0)),
            pl.BlockSpec((D, D), lambda i: (0, 0)),
        ],
        out_specs=pl.BlockSpec((blk, D), lambda i: (i, 0)),
        out_shape=jax.ShapeDtypeStruct((N, D), jnp.float32),
    )(psum, pcnt3, w.astype(jnp.bfloat16))


def kernel(feat, edge_index, W_neigh):
    ei = edge_index.reshape(2, NC * NS * NGROUP, GCHUNKS, CHUNK)
    psum, pcnt = _sc_agg(feat, ei)
    return _tc_finish(psum, pcnt.reshape(NC, NPAD, 1), W_neigh)


# speedup vs baseline: 1.1507x; 1.0394x over previous
import jax
import jax.numpy as jnp
from jax import lax
from jax.experimental import pallas as pl
from jax.experimental.pallas import tpu as pltpu
from jax.experimental.pallas import tpu_sc as plsc

N = 10000
D = 128
E = 320000
NC = 2
NS = 16
NPAD = 10240
ROWS_PER_TILE = NPAD // NS
CHUNK = 80
CHUNKS_PER_TILE = E // (NC * NS * CHUNK)
NGROUP = 5
GCHUNKS = CHUNKS_PER_TILE // NGROUP
ZROWS = 16
NBUF = 3


def _sc_agg_body(feat_hbm, ei_hbm, psum_hbm, pcnt_hbm,
                 acc_sh, cnt_sh, sidx, didx, rows0, rows1, rows2,
                 ones_v, zrows, zcnt, semg, sems):
    rows = (rows0, rows1, rows2)
    c = lax.axis_index("core")
    s = lax.axis_index("subcore")
    t = c * NS + s

    @pl.loop(0, CHUNK // 16)
    def _(k):
        ones_v[pl.ds(k * 16, 16)] = jnp.ones((16,), jnp.float32)

    @pl.loop(0, ZROWS)
    def _(r):
        @pl.loop(0, D // 16)
        def _(k):
            zrows[r, pl.ds(k * 16, 16)] = jnp.zeros((16,), jnp.float32)

    @pl.loop(0, ROWS_PER_TILE // 16)
    def _(k):
        zcnt[pl.ds(k * 16, 16)] = jnp.zeros((16,), jnp.float32)

    @pl.loop(0, ROWS_PER_TILE // ZROWS)
    def _(j):
        off = s * ROWS_PER_TILE + j * ZROWS
        pltpu.async_copy(zrows, acc_sh.at[pl.ds(off, ZROWS)], semg.at[0])
    pltpu.async_copy(zcnt, cnt_sh.at[pl.ds(s * ROWS_PER_TILE, ROWS_PER_TILE)],
                     semg.at[1])

    @pl.loop(0, ROWS_PER_TILE // ZROWS)
    def _(j):
        pltpu.make_async_copy(zrows, acc_sh.at[pl.ds(s * ROWS_PER_TILE,
                                                     ZROWS)],
                              semg.at[0]).wait()
    pltpu.make_async_copy(zcnt, cnt_sh.at[pl.ds(s * ROWS_PER_TILE,
                                                ROWS_PER_TILE)],
                          semg.at[1]).wait()

    plsc.subcore_barrier()

    @pl.loop(0, NGROUP)
    def _(g):
        plane = t * NGROUP + g
        pltpu.async_copy(ei_hbm.at[0, plane], sidx, semg.at[0])
        pltpu.async_copy(ei_hbm.at[1, plane], didx, semg.at[1])
        pltpu.make_async_copy(ei_hbm.at[0, plane], sidx, semg.at[0]).wait()
        pltpu.make_async_copy(ei_hbm.at[1, plane], didx, semg.at[1]).wait()

        for b in range(NBUF):
            pltpu.async_copy(feat_hbm.at[sidx.at[b]], rows[b], semg.at[b])

        @pl.loop(0, GCHUNKS, step=NBUF)
        def _(i):
            for b in range(NBUF):
                j = i + b

                @pl.when(j < GCHUNKS)
                def _():
                    pltpu.make_async_copy(feat_hbm.at[sidx.at[j]], rows[b],
                                          semg.at[b]).wait()
                    pltpu.async_copy(rows[b], acc_sh.at[didx.at[j]],
                                     sems.at[b], add=True)
                    pltpu.sync_copy(ones_v, cnt_sh.at[didx.at[j]], add=True)

                @pl.when(j + NBUF < GCHUNKS)
                def _():
                    pltpu.make_async_copy(rows[b], acc_sh.at[didx.at[j]],
                                          sems.at[b]).wait()
                    pltpu.async_copy(feat_hbm.at[sidx.at[j + NBUF]], rows[b],
                                     semg.at[b])

        for b in range(NBUF):
            pltpu.make_async_copy(rows[b], acc_sh.at[didx.at[0]],
                                  sems.at[b]).wait()

    plsc.subcore_barrier()

    off = s * ROWS_PER_TILE
    pltpu.async_copy(acc_sh.at[pl.ds(off, ROWS_PER_TILE)],
                     psum_hbm.at[c, pl.ds(off, ROWS_PER_TILE)], semg.at[0])
    pltpu.async_copy(cnt_sh.at[pl.ds(off, ROWS_PER_TILE)],
                     pcnt_hbm.at[c, pl.ds(off, ROWS_PER_TILE)], semg.at[1])
    pltpu.make_async_copy(acc_sh.at[pl.ds(off, ROWS_PER_TILE)],
                          psum_hbm.at[c, pl.ds(off, ROWS_PER_TILE)],
                          semg.at[0]).wait()
    pltpu.make_async_copy(cnt_sh.at[pl.ds(off, ROWS_PER_TILE)],
                          pcnt_hbm.at[c, pl.ds(off, ROWS_PER_TILE)],
                          semg.at[1]).wait()


def _sc_agg(feat, ei):
    mesh = plsc.VectorSubcoreMesh(core_axis_name="core",
                                  subcore_axis_name="subcore")
    f = pl.kernel(
        _sc_agg_body,
        out_type=[
            jax.ShapeDtypeStruct((NC, NPAD, D), jnp.float32),
            jax.ShapeDtypeStruct((NC, NPAD), jnp.float32),
        ],
        mesh=mesh,
        scratch_types=[
            pltpu.VMEM_SHARED((NPAD, D), jnp.float32),
            pltpu.VMEM_SHARED((NPAD,), jnp.float32),
            pltpu.VMEM((GCHUNKS, CHUNK), jnp.int32),
            pltpu.VMEM((GCHUNKS, CHUNK), jnp.int32),
            pltpu.VMEM((CHUNK, D), jnp.float32),
            pltpu.VMEM((CHUNK, D), jnp.float32),
            pltpu.VMEM((CHUNK, D), jnp.float32),
            pltpu.VMEM((CHUNK,), jnp.float32),
            pltpu.VMEM((ZROWS, D), jnp.float32),
            pltpu.VMEM((ROWS_PER_TILE,), jnp.float32),
            pltpu.SemaphoreType.DMA((NBUF,)),
            pltpu.SemaphoreType.DMA((NBUF,)),
        ],
    )
    return f(feat, ei)


def _tc_body(ps_ref, pc_ref, w_ref, o_ref):
    p = ps_ref[...]
    ssum = p[0] + p[1]
    cc = pc_ref[...]
    deg = jnp.maximum(cc[0] + cc[1], 1.0)
    h = (ssum / deg).astype(jnp.bfloat16)
    o_ref[...] = lax.dot_general(h, w_ref[...], (((1,), (1,)), ((), ())),
                                 preferred_element_type=jnp.float32)


def _tc_finish(psum, pcnt3, w):
    blk = 2000
    return pl.pallas_call(
        _tc_body,
        grid=(N // blk,),
        in_specs=[
            pl.BlockSpec((NC, blk, D), lambda i: (0, i, 0)),
            pl.BlockSpec((NC, blk, 1), lambda i: (0, i, 0)),
            pl.BlockSpec((D, D), lambda i: (0, 0)),
        ],
        out_specs=pl.BlockSpec((blk, D), lambda i: (i, 0)),
        out_shape=jax.ShapeDtypeStruct((N, D), jnp.float32),
    )(psum, pcnt3, w.astype(jnp.bfloat16))


def kernel(feat, edge_index, W_neigh):
    ei = edge_index.reshape(2, NC * NS * NGROUP, GCHUNKS, CHUNK)
    psum, pcnt = _sc_agg(feat, ei)
    return _tc_finish(psum, pcnt.reshape(NC, NPAD, 1), W_neigh)


# speedup vs baseline: 1.1591x; 1.0073x over previous
import jax
import jax.numpy as jnp
from jax import lax
from jax.experimental import pallas as pl
from jax.experimental.pallas import tpu as pltpu
from jax.experimental.pallas import tpu_sc as plsc

N = 10000
D = 128
E = 320000
NC = 2
NS = 16
NPAD = 10240
ROWS_PER_TILE = NPAD // NS
CHUNK = 80
CHUNKS_PER_TILE = E // (NC * NS * CHUNK)
NGROUP = 5
GCHUNKS = CHUNKS_PER_TILE // NGROUP
ZROWS = 16
NBUF = 3


def _sc_agg_body(feat_hbm, ei_hbm, psum_hbm, pcnt_hbm,
                 acc_sh, cnt_sh, sidx, didx, rows0, rows1, rows2,
                 ones_v, zrows, zcnt, semg, sems):
    rows = (rows0, rows1, rows2)
    c = lax.axis_index("core")
    s = lax.axis_index("subcore")
    t = c * NS + s

    @pl.loop(0, CHUNK // 16)
    def _(k):
        ones_v[pl.ds(k * 16, 16)] = jnp.ones((16,), jnp.float32)

    @pl.loop(0, ZROWS)
    def _(r):
        @pl.loop(0, D // 16)
        def _(k):
            zrows[r, pl.ds(k * 16, 16)] = jnp.zeros((16,), jnp.float32)

    @pl.loop(0, ROWS_PER_TILE // 16)
    def _(k):
        zcnt[pl.ds(k * 16, 16)] = jnp.zeros((16,), jnp.float32)

    @pl.loop(0, ROWS_PER_TILE // ZROWS)
    def _(j):
        off = s * ROWS_PER_TILE + j * ZROWS
        pltpu.async_copy(zrows, acc_sh.at[pl.ds(off, ZROWS)], sems.at[0])
    pltpu.async_copy(zcnt, cnt_sh.at[pl.ds(s * ROWS_PER_TILE, ROWS_PER_TILE)],
                     sems.at[1])

    @pl.loop(0, ROWS_PER_TILE // ZROWS)
    def _(j):
        pltpu.make_async_copy(zrows, acc_sh.at[pl.ds(s * ROWS_PER_TILE,
                                                     ZROWS)],
                              sems.at[0]).wait()
    pltpu.make_async_copy(zcnt, cnt_sh.at[pl.ds(s * ROWS_PER_TILE,
                                                ROWS_PER_TILE)],
                          sems.at[1]).wait()

    plane0 = t * NGROUP
    pltpu.async_copy(ei_hbm.at[0, plane0], sidx, semg.at[0])
    pltpu.async_copy(ei_hbm.at[1, plane0], didx, semg.at[1])
    pltpu.make_async_copy(ei_hbm.at[0, plane0], sidx, semg.at[0]).wait()
    pltpu.make_async_copy(ei_hbm.at[1, plane0], didx, semg.at[1]).wait()
    for b in range(NBUF):
        pltpu.async_copy(feat_hbm.at[sidx.at[b]], rows[b], semg.at[b])

    plsc.subcore_barrier()

    @pl.loop(0, NGROUP)
    def _(g):
        @pl.when(g > 0)
        def _():
            plane = t * NGROUP + g
            pltpu.async_copy(ei_hbm.at[0, plane], sidx, semg.at[0])
            pltpu.async_copy(ei_hbm.at[1, plane], didx, semg.at[1])
            pltpu.make_async_copy(ei_hbm.at[0, plane], sidx,
                                  semg.at[0]).wait()
            pltpu.make_async_copy(ei_hbm.at[1, plane], didx,
                                  semg.at[1]).wait()
            for b in range(NBUF):
                pltpu.async_copy(feat_hbm.at[sidx.at[b]], rows[b],
                                 semg.at[b])

        @pl.loop(0, GCHUNKS, step=NBUF)
        def _(i):
            for b in range(NBUF):
                j = i + b

                @pl.when(j < GCHUNKS)
                def _():
                    pltpu.make_async_copy(feat_hbm.at[sidx.at[j]], rows[b],
                                          semg.at[b]).wait()
                    pltpu.async_copy(rows[b], acc_sh.at[didx.at[j]],
                                     sems.at[b], add=True)
                    pltpu.sync_copy(ones_v, cnt_sh.at[didx.at[j]], add=True)

                @pl.when(j + NBUF < GCHUNKS)
                def _():
                    pltpu.make_async_copy(rows[b], acc_sh.at[didx.at[j]],
                                          sems.at[b]).wait()
                    pltpu.async_copy(feat_hbm.at[sidx.at[j + NBUF]], rows[b],
                                     semg.at[b])

        for b in range(NBUF):
            pltpu.make_async_copy(rows[b], acc_sh.at[didx.at[0]],
                                  sems.at[b]).wait()

    plsc.subcore_barrier()

    off = s * ROWS_PER_TILE
    pltpu.async_copy(acc_sh.at[pl.ds(off, ROWS_PER_TILE)],
                     psum_hbm.at[c, pl.ds(off, ROWS_PER_TILE)], semg.at[0])
    pltpu.async_copy(cnt_sh.at[pl.ds(off, ROWS_PER_TILE)],
                     pcnt_hbm.at[c, pl.ds(off, ROWS_PER_TILE)], semg.at[1])
    pltpu.make_async_copy(acc_sh.at[pl.ds(off, ROWS_PER_TILE)],
                          psum_hbm.at[c, pl.ds(off, ROWS_PER_TILE)],
                          semg.at[0]).wait()
    pltpu.make_async_copy(cnt_sh.at[pl.ds(off, ROWS_PER_TILE)],
                          pcnt_hbm.at[c, pl.ds(off, ROWS_PER_TILE)],
                          semg.at[1]).wait()


def _sc_agg(feat, ei):
    mesh = plsc.VectorSubcoreMesh(core_axis_name="core",
                                  subcore_axis_name="subcore")
    f = pl.kernel(
        _sc_agg_body,
        out_type=[
            jax.ShapeDtypeStruct((NC, NPAD, D), jnp.float32),
            jax.ShapeDtypeStruct((NC, NPAD), jnp.float32),
        ],
        mesh=mesh,
        scratch_types=[
            pltpu.VMEM_SHARED((NPAD, D), jnp.float32),
            pltpu.VMEM_SHARED((NPAD,), jnp.float32),
            pltpu.VMEM((GCHUNKS, CHUNK), jnp.int32),
            pltpu.VMEM((GCHUNKS, CHUNK), jnp.int32),
            pltpu.VMEM((CHUNK, D), jnp.float32),
            pltpu.VMEM((CHUNK, D), jnp.float32),
            pltpu.VMEM((CHUNK, D), jnp.float32),
            pltpu.VMEM((CHUNK,), jnp.float32),
            pltpu.VMEM((ZROWS, D), jnp.float32),
            pltpu.VMEM((ROWS_PER_TILE,), jnp.float32),
            pltpu.SemaphoreType.DMA((NBUF,)),
            pltpu.SemaphoreType.DMA((NBUF,)),
        ],
    )
    return f(feat, ei)


def _tc_body(ps_ref, pc_ref, w_ref, o_ref):
    p = ps_ref[...]
    ssum = p[0] + p[1]
    cc = pc_ref[...]
    deg = jnp.maximum(cc[0] + cc[1], 1.0)
    h = (ssum / deg).astype(jnp.bfloat16)
    o_ref[...] = lax.dot_general(h, w_ref[...], (((1,), (1,)), ((), ())),
                                 preferred_element_type=jnp.float32)


def _tc_finish(psum, pcnt3, w):
    blk = 5000
    return pl.pallas_call(
        _tc_body,
        grid=(N // blk,),
        in_specs=[
            pl.BlockSpec((NC, blk, D), lambda i: (0, i, 0)),
            pl.BlockSpec((NC, blk, 1), lambda i: (0, i, 0)),
            pl.BlockSpec((D, D), lambda i: (0, 0)),
        ],
        out_specs=pl.BlockSpec((blk, D), lambda i: (i, 0)),
        out_shape=jax.ShapeDtypeStruct((N, D), jnp.float32),
    )(psum, pcnt3, w.astype(jnp.bfloat16))


def kernel(feat, edge_index, W_neigh):
    ei = edge_index.reshape(2, NC * NS * NGROUP, GCHUNKS, CHUNK)
    psum, pcnt = _sc_agg(feat, ei)
    return _tc_finish(psum, pcnt.reshape(NC, NPAD, 1), W_neigh)
